# baseline (speedup 1.0000x reference)
import functools

import jax
import jax.numpy as jnp
from jax import lax
from jax.experimental import pallas as pl
from jax.experimental.pallas import tpu as pltpu
from jax.experimental.pallas import tpu_sc as plsc

N_NODES = 10000
N_EDGES = 320000
D_FEAT = 128
NUM_CLASS = 2

NC, NS, L = 2, 16, 16
NW = NC * NS
P_ROWS = 2 * NUM_CLASS

BLK = 128
N_BLOCKS = N_EDGES // BLK
BLOCKS_PER = N_BLOCKS // NW
EXTRA_TILES = N_BLOCKS - BLOCKS_PER * NW
CHUNKB = 13
N_CHUNKS = BLOCKS_PER // CHUNKB
E_MAIN = BLOCKS_PER * BLK
E_STAGE = E_MAIN + BLK
G_CHUNK = CHUNKB * BLK // L
G_EXTRA = BLK // L


def _proj_body(x_ref, w_ref, b_ref, p_ref):
    wct = jnp.concatenate([w_ref[:, :D_FEAT], w_ref[:, D_FEAT:]], axis=0)
    bcol = jnp.concatenate(
        [b_ref[...], jnp.zeros((NUM_CLASS, 1), jnp.float32)], axis=0
    )
    p_ref[...] = (
        lax.dot_general(
            wct,
            x_ref[...],
            (((1,), (1,)), ((), ())),
            preferred_element_type=jnp.float32,
        )
        + bcol
    )


_proj = pl.pallas_call(
    _proj_body,
    out_shape=jax.ShapeDtypeStruct((P_ROWS, N_NODES), jnp.float32),
)

_mesh = plsc.VectorSubcoreMesh(
    core_axis_name="c", subcore_axis_name="s", num_cores=NC, num_subcores=NS
)


@functools.partial(
    pl.kernel,
    out_type=jax.ShapeDtypeStruct((N_BLOCKS, NUM_CLASS, BLK), jnp.float32),
    mesh=_mesh,
    scratch_types=[
        pltpu.VMEM((P_ROWS, N_NODES), jnp.float32),
        pltpu.VMEM((BLOCKS_PER + 1, BLK), jnp.int32),
        pltpu.VMEM((BLOCKS_PER + 1, BLK), jnp.int32),
        pltpu.VMEM((2, CHUNKB, NUM_CLASS, BLK), jnp.float32),
        pltpu.VMEM((1, NUM_CLASS, BLK), jnp.float32),
        pltpu.SemaphoreType.DMA,
        pltpu.SemaphoreType.DMA,
        pltpu.SemaphoreType.DMA,
    ],
    compiler_params=pltpu.CompilerParams(
        needs_layout_passes=False, use_tc_tiling_on_sc=False
    ),
)
def _edge_score(
    p_hbm, ei_hbm, out_hbm, p_v, src_v, dst_v, out_v, ext_v, sem_in, sem_p, sem_out
):
    wid = lax.axis_index("s") * NC + lax.axis_index("c")
    has_extra = wid < EXTRA_TILES
    bbase = wid * BLOCKS_PER + jnp.minimum(wid, EXTRA_TILES)

    cp_p = pltpu.async_copy(p_hbm, p_v, sem_p)
    cp_s = pltpu.async_copy(
        ei_hbm.at[pl.ds(bbase, BLOCKS_PER), 0],
        src_v.at[pl.ds(0, BLOCKS_PER)],
        sem_in,
    )
    cp_d = pltpu.async_copy(
        ei_hbm.at[pl.ds(bbase, BLOCKS_PER), 1],
        dst_v.at[pl.ds(0, BLOCKS_PER)],
        sem_in,
    )

    @pl.when(has_extra)
    def _():
        pltpu.async_copy(
            ei_hbm.at[pl.ds(bbase + BLOCKS_PER, 1), 0],
            src_v.at[pl.ds(BLOCKS_PER, 1)],
            sem_in,
        ).wait()
        pltpu.async_copy(
            ei_hbm.at[pl.ds(bbase + BLOCKS_PER, 1), 1],
            dst_v.at[pl.ds(BLOCKS_PER, 1)],
            sem_in,
        ).wait()

    cp_s.wait()
    cp_d.wait()
    cp_p.wait()

    lane = lax.iota(jnp.int32, L)
    r0 = jnp.zeros((L,), jnp.int32)
    r1 = r0 + 1
    r2 = r0 + 2
    r3 = r0 + 3

    def do_group(b_local, jbase, buf_ref, b_buf):
        s = src_v[b_local, pl.ds(jbase, L)]
        d = dst_v[b_local, pl.ds(jbase, L)]
        o0 = plsc.load_gather(p_v, [r0, s]) + plsc.load_gather(p_v, [r2, d])
        o1 = plsc.load_gather(p_v, [r1, s]) + plsc.load_gather(p_v, [r3, d])
        bv = r0 + b_buf
        jv = lane + jbase
        plsc.store_scatter(buf_ref, [bv, r0, jv], o0)
        plsc.store_scatter(buf_ref, [bv, r1, jv], o1)

    out_cp = [None, None]
    for k in range(N_CHUNKS):
        buf = k % 2
        if out_cp[buf] is not None:
            out_cp[buf].wait()

        def group(g, carry):
            bc = g // (BLK // L)
            jbase = pl.multiple_of((g % (BLK // L)) * L, L)
            do_group(k * CHUNKB + bc, jbase, out_v.at[buf], bc)
            return carry

        lax.fori_loop(0, G_CHUNK, group, 0, unroll=8)
        out_cp[buf] = pltpu.async_copy(
            out_v.at[buf], out_hbm.at[pl.ds(bbase + k * CHUNKB, CHUNKB)], sem_out
        )

    @pl.when(has_extra)
    def _():
        def group(g, carry):
            do_group(BLOCKS_PER, pl.multiple_of(g * L, L), ext_v, 0)
            return carry

        lax.fori_loop(0, G_EXTRA, group, 0, unroll=4)
        pltpu.async_copy(
            ext_v, out_hbm.at[pl.ds(bbase + BLOCKS_PER, 1)], sem_out
        ).wait()

    for cp in out_cp:
        if cp is not None:
            cp.wait()


def kernel(x, edge_index, W, b):
    ei3d = (
        edge_index.astype(jnp.int32)
        .reshape(2, N_BLOCKS, BLK)
        .transpose(1, 0, 2)
    )
    p = _proj(x, W, b.reshape(NUM_CLASS, 1))
    out3d = _edge_score(p, ei3d)
    return out3d.transpose(0, 2, 1).reshape(N_EDGES, NUM_CLASS)


# speedup vs baseline: 1.0365x; 1.0365x over previous
import functools

import jax
import jax.numpy as jnp
from jax import lax
from jax.experimental import pallas as pl
from jax.experimental.pallas import tpu as pltpu
from jax.experimental.pallas import tpu_sc as plsc

N_NODES = 10000
N_EDGES = 320000
D_FEAT = 128
NUM_CLASS = 2

NC, NS, L = 2, 16, 16
NW = NC * NS
P_ROWS = 2 * NUM_CLASS

BLK = 128
N_BLOCKS = N_EDGES // BLK
BLOCKS_PER = N_BLOCKS // NW
EXTRA_TILES = N_BLOCKS - BLOCKS_PER * NW
CHUNKB = 13
N_CHUNKS = BLOCKS_PER // CHUNKB
E_MAIN = BLOCKS_PER * BLK
E_STAGE = E_MAIN + BLK
G_CHUNK = CHUNKB * BLK // L
G_EXTRA = BLK // L


def _proj_body(x_ref, w_ref, b_ref, p_ref):
    wct = jnp.concatenate([w_ref[:, :D_FEAT], w_ref[:, D_FEAT:]], axis=0)
    bcol = jnp.concatenate(
        [b_ref[...].T, jnp.zeros((NUM_CLASS, 1), jnp.float32)], axis=0
    )
    p_ref[...] = (
        lax.dot_general(
            wct,
            x_ref[...],
            (((1,), (1,)), ((), ())),
            preferred_element_type=jnp.float32,
        )
        + bcol
    )


_proj = pl.pallas_call(
    _proj_body,
    out_shape=jax.ShapeDtypeStruct((P_ROWS, N_NODES), jnp.float32),
)

_mesh = plsc.VectorSubcoreMesh(
    core_axis_name="c", subcore_axis_name="s", num_cores=NC, num_subcores=NS
)


@functools.partial(
    pl.kernel,
    out_type=jax.ShapeDtypeStruct((N_BLOCKS, NUM_CLASS, BLK), jnp.float32),
    mesh=_mesh,
    scratch_types=[
        pltpu.VMEM((P_ROWS, N_NODES), jnp.float32),
        pltpu.VMEM((BLOCKS_PER + 1, BLK), jnp.int32),
        pltpu.VMEM((BLOCKS_PER + 1, BLK), jnp.int32),
        pltpu.VMEM((2, CHUNKB, NUM_CLASS, BLK), jnp.float32),
        pltpu.VMEM((1, NUM_CLASS, BLK), jnp.float32),
        pltpu.SemaphoreType.DMA,
        pltpu.SemaphoreType.DMA,
        pltpu.SemaphoreType.DMA,
    ],
    compiler_params=pltpu.CompilerParams(
        needs_layout_passes=False, use_tc_tiling_on_sc=False
    ),
)
def _edge_score(
    p_hbm, ei_hbm, out_hbm, p_v, src_v, dst_v, out_v, ext_v, sem_in, sem_p, sem_out
):
    wid = lax.axis_index("s") * NC + lax.axis_index("c")
    has_extra = wid < EXTRA_TILES
    bbase = wid * BLOCKS_PER + jnp.minimum(wid, EXTRA_TILES)

    cp_p = pltpu.async_copy(p_hbm, p_v, sem_p)
    cp_s = pltpu.async_copy(
        ei_hbm.at[pl.ds(bbase, BLOCKS_PER), 0],
        src_v.at[pl.ds(0, BLOCKS_PER)],
        sem_in,
    )
    cp_d = pltpu.async_copy(
        ei_hbm.at[pl.ds(bbase, BLOCKS_PER), 1],
        dst_v.at[pl.ds(0, BLOCKS_PER)],
        sem_in,
    )

    @pl.when(has_extra)
    def _():
        pltpu.async_copy(
            ei_hbm.at[pl.ds(bbase + BLOCKS_PER, 1), 0],
            src_v.at[pl.ds(BLOCKS_PER, 1)],
            sem_in,
        ).wait()
        pltpu.async_copy(
            ei_hbm.at[pl.ds(bbase + BLOCKS_PER, 1), 1],
            dst_v.at[pl.ds(BLOCKS_PER, 1)],
            sem_in,
        ).wait()

    cp_s.wait()
    cp_d.wait()
    cp_p.wait()

    lane = lax.iota(jnp.int32, L)
    r0 = jnp.zeros((L,), jnp.int32)
    r1 = r0 + 1
    r2 = r0 + 2
    r3 = r0 + 3

    def do_group(b_local, jbase, buf_ref, b_buf):
        s = src_v[b_local, pl.ds(jbase, L)]
        d = dst_v[b_local, pl.ds(jbase, L)]
        o0 = plsc.load_gather(p_v, [r0, s]) + plsc.load_gather(p_v, [r2, d])
        o1 = plsc.load_gather(p_v, [r1, s]) + plsc.load_gather(p_v, [r3, d])
        bv = r0 + b_buf
        jv = lane + jbase
        plsc.store_scatter(buf_ref, [bv, r0, jv], o0)
        plsc.store_scatter(buf_ref, [bv, r1, jv], o1)

    out_cp = [None, None]
    for k in range(N_CHUNKS):
        buf = k % 2
        if out_cp[buf] is not None:
            out_cp[buf].wait()

        def group(g, carry):
            bc = g // (BLK // L)
            jbase = pl.multiple_of((g % (BLK // L)) * L, L)
            do_group(k * CHUNKB + bc, jbase, out_v.at[buf], bc)
            return carry

        lax.fori_loop(0, G_CHUNK, group, 0, unroll=8)
        out_cp[buf] = pltpu.async_copy(
            out_v.at[buf], out_hbm.at[pl.ds(bbase + k * CHUNKB, CHUNKB)], sem_out
        )

    @pl.when(has_extra)
    def _():
        def group(g, carry):
            do_group(BLOCKS_PER, pl.multiple_of(g * L, L), ext_v, 0)
            return carry

        lax.fori_loop(0, G_EXTRA, group, 0, unroll=4)
        pltpu.async_copy(
            ext_v, out_hbm.at[pl.ds(bbase + BLOCKS_PER, 1)], sem_out
        ).wait()

    for cp in out_cp:
        if cp is not None:
            cp.wait()


def kernel(x, edge_index, W, b):
    ei3d = (
        edge_index.astype(jnp.int32)
        .reshape(2, N_BLOCKS, BLK)
        .transpose(1, 0, 2)
    )
    p = _proj(x, W, b.reshape(1, NUM_CLASS))
    out3d = _edge_score(p, ei3d)
    return out3d.transpose(0, 2, 1).reshape(N_EDGES, NUM_CLASS)


# speedup vs baseline: 1.2292x; 1.1859x over previous
import functools

import jax
import jax.numpy as jnp
from jax import lax
from jax.experimental import pallas as pl
from jax.experimental.pallas import tpu as pltpu
from jax.experimental.pallas import tpu_sc as plsc

N_NODES = 10000
N_EDGES = 320000
D_FEAT = 128
NUM_CLASS = 2

NC, NS, L = 2, 16, 16
NW = NC * NS
P_ROWS = 2 * NUM_CLASS

BLK = 128
N_BLOCKS = N_EDGES // BLK
BLOCKS_PER = N_BLOCKS // NW
EXTRA_TILES = N_BLOCKS - BLOCKS_PER * NW
CHUNKB = 13
N_CHUNKS = BLOCKS_PER // CHUNKB
E_MAIN = BLOCKS_PER * BLK
E_STAGE = E_MAIN + BLK
G_CHUNK = CHUNKB * BLK // L
G_EXTRA = BLK // L


def _proj_body(x_ref, w_ref, b_ref, p_ref):
    wct = jnp.concatenate([w_ref[:, :D_FEAT], w_ref[:, D_FEAT:]], axis=0)
    bcol = jnp.concatenate(
        [b_ref[...].T, jnp.zeros((NUM_CLASS, 1), jnp.float32)], axis=0
    )
    p_ref[...] = (
        lax.dot_general(
            wct,
            x_ref[...],
            (((1,), (1,)), ((), ())),
            preferred_element_type=jnp.float32,
        )
        + bcol
    )


_proj = pl.pallas_call(
    _proj_body,
    out_shape=jax.ShapeDtypeStruct((P_ROWS, N_NODES), jnp.float32),
)

_mesh = plsc.VectorSubcoreMesh(
    core_axis_name="c", subcore_axis_name="s", num_cores=NC, num_subcores=NS
)


@functools.partial(
    pl.kernel,
    out_type=jax.ShapeDtypeStruct((N_BLOCKS, NUM_CLASS, BLK), jnp.float32),
    mesh=_mesh,
    scratch_types=[
        pltpu.VMEM((P_ROWS, N_NODES), jnp.float32),
        pltpu.VMEM((BLOCKS_PER + 1, BLK), jnp.int32),
        pltpu.VMEM((BLOCKS_PER + 1, BLK), jnp.int32),
        pltpu.VMEM((2, CHUNKB, NUM_CLASS, BLK), jnp.float32),
        pltpu.VMEM((1, NUM_CLASS, BLK), jnp.float32),
        pltpu.SemaphoreType.DMA,
        pltpu.SemaphoreType.DMA,
        pltpu.SemaphoreType.DMA,
    ],
    compiler_params=pltpu.CompilerParams(
        needs_layout_passes=False, use_tc_tiling_on_sc=False
    ),
)
def _edge_score(
    p_hbm, ei_hbm, out_hbm, p_v, src_v, dst_v, out_v, ext_v, sem_in, sem_p, sem_out
):
    wid = lax.axis_index("s") * NC + lax.axis_index("c")
    has_extra = wid < EXTRA_TILES
    bbase = wid * BLOCKS_PER + jnp.minimum(wid, EXTRA_TILES)

    cp_p = pltpu.async_copy(p_hbm, p_v, sem_p)
    cp_s = pltpu.async_copy(
        ei_hbm.at[pl.ds(bbase, BLOCKS_PER), 0],
        src_v.at[pl.ds(0, BLOCKS_PER)],
        sem_in,
    )
    cp_d = pltpu.async_copy(
        ei_hbm.at[pl.ds(bbase, BLOCKS_PER), 1],
        dst_v.at[pl.ds(0, BLOCKS_PER)],
        sem_in,
    )

    @pl.when(has_extra)
    def _():
        pltpu.async_copy(
            ei_hbm.at[pl.ds(bbase + BLOCKS_PER, 1), 0],
            src_v.at[pl.ds(BLOCKS_PER, 1)],
            sem_in,
        ).wait()
        pltpu.async_copy(
            ei_hbm.at[pl.ds(bbase + BLOCKS_PER, 1), 1],
            dst_v.at[pl.ds(BLOCKS_PER, 1)],
            sem_in,
        ).wait()

    cp_s.wait()
    cp_d.wait()
    cp_p.wait()

    lane = lax.iota(jnp.int32, L)
    r0 = jnp.zeros((L,), jnp.int32)
    r1 = r0 + 1
    r2 = r0 + 2
    r3 = r0 + 3

    def do_group(b_local, jbase, buf_ref, b_buf):
        s = src_v[b_local, pl.ds(jbase, L)]
        d = dst_v[b_local, pl.ds(jbase, L)]
        o0 = plsc.load_gather(p_v, [r0, s]) + plsc.load_gather(p_v, [r2, d])
        o1 = plsc.load_gather(p_v, [r1, s]) + plsc.load_gather(p_v, [r3, d])
        bv = r0 + b_buf
        jv = lane + jbase
        plsc.store_scatter(buf_ref, [bv, r0, jv], o0)
        plsc.store_scatter(buf_ref, [bv, r1, jv], o1)

    out_cp = [None, None]
    for k in range(N_CHUNKS):
        buf = k % 2
        if out_cp[buf] is not None:
            out_cp[buf].wait()

        @plsc.parallel_loop(0, G_CHUNK, unroll=4)
        def _(g):
            bc = g // (BLK // L)
            jbase = pl.multiple_of((g % (BLK // L)) * L, L)
            do_group(k * CHUNKB + bc, jbase, out_v.at[buf], bc)
        out_cp[buf] = pltpu.async_copy(
            out_v.at[buf], out_hbm.at[pl.ds(bbase + k * CHUNKB, CHUNKB)], sem_out
        )

    @pl.when(has_extra)
    def _():
        @plsc.parallel_loop(0, G_EXTRA, unroll=4)
        def _(g):
            do_group(BLOCKS_PER, pl.multiple_of(g * L, L), ext_v, 0)
        pltpu.async_copy(
            ext_v, out_hbm.at[pl.ds(bbase + BLOCKS_PER, 1)], sem_out
        ).wait()

    for cp in out_cp:
        if cp is not None:
            cp.wait()


def kernel(x, edge_index, W, b):
    ei3d = (
        edge_index.astype(jnp.int32)
        .reshape(2, N_BLOCKS, BLK)
        .transpose(1, 0, 2)
    )
    p = _proj(x, W, b.reshape(1, NUM_CLASS))
    out3d = _edge_score(p, ei3d)
    return out3d.transpose(0, 2, 1).reshape(N_EDGES, NUM_CLASS)
